# baseline (speedup 1.0000x reference)
import functools

import jax
import jax.numpy as jnp
from jax import lax
from jax.experimental import pallas as pl
from jax.experimental.pallas import tpu as pltpu
from jax.experimental.pallas import tpu_sc as plsc

N = 100000
D = 128
G = 128
NC, NS = 2, 16
NW = NC * NS
GPW = 25
RPW = GPW * G
NPAD = NW * RPW
NFULL = N // G
TAIL = N - NFULL * G

_mesh = plsc.VectorSubcoreMesh(core_axis_name="c", subcore_axis_name="s")


@functools.partial(
    pl.kernel,
    mesh=_mesh,
    out_type=jax.ShapeDtypeStruct((N, D), jnp.float32),
    scratch_types=[
        pltpu.VMEM((RPW,), jnp.int32),
        pltpu.VMEM((G, D), jnp.float32),
        pltpu.SemaphoreType.DMA,
    ],
)
def _emb_lookup(idx_hbm, table_hbm, out_hbm, idx_v, rows_v, sem):
    wid = lax.axis_index("s") * NC + lax.axis_index("c")
    pltpu.sync_copy(idx_hbm.at[pl.ds(pl.multiple_of(wid * RPW, G), RPW)], idx_v)

    def body(g, _):
        chunk = wid * GPW + g
        row0 = pl.multiple_of(chunk * G, G)
        goff = pl.multiple_of(g * G, G)

        @pl.when(chunk < NFULL)
        def _full():
            pltpu.async_copy(table_hbm.at[idx_v.at[pl.ds(goff, G)]],
                             rows_v, sem).wait()
            pltpu.sync_copy(rows_v, out_hbm.at[pl.ds(row0, G)])

        @pl.when(chunk == NFULL)
        def _partial():
            pltpu.async_copy(table_hbm.at[idx_v.at[pl.ds(goff, G)]],
                             rows_v, sem).wait()
            pltpu.sync_copy(rows_v.at[pl.ds(0, TAIL)],
                            out_hbm.at[pl.ds(row0, TAIL)])

        return ()

    lax.fori_loop(0, GPW, body, ())


def kernel(x, table):
    idx = x[:, 0].astype(jnp.int32)
    idx = jnp.pad(idx, (0, NPAD - N))
    return _emb_lookup(idx, table)


# speedup vs baseline: 4.4037x; 4.4037x over previous
import functools

import jax
import jax.numpy as jnp
from jax import lax
from jax.experimental import pallas as pl
from jax.experimental.pallas import tpu as pltpu
from jax.experimental.pallas import tpu_sc as plsc

N = 100000
D = 128
G = 128
NC, NS = 2, 16
NW = NC * NS
GPW = 25
RPW = GPW * G
NPAD = NW * RPW
NFULL = N // G
TAIL = N - NFULL * G

_mesh = plsc.VectorSubcoreMesh(core_axis_name="c", subcore_axis_name="s")


@functools.partial(
    pl.kernel,
    mesh=_mesh,
    out_type=jax.ShapeDtypeStruct((N, D), jnp.float32),
    scratch_types=[
        pltpu.VMEM((RPW,), jnp.int32),
        pltpu.VMEM((G, D), jnp.float32),
        pltpu.VMEM_SHARED((28, D), jnp.float32),
        pltpu.SemaphoreType.DMA,
    ],
)
def _emb_lookup(idx_hbm, table_hbm, out_hbm, idx_v, rows_v, table_v, sem):
    sid = lax.axis_index("s")
    wid = sid * NC + lax.axis_index("c")

    @pl.when(sid == 0)
    def _stage_table():
        pltpu.sync_copy(table_hbm, table_v)

    plsc.subcore_barrier()
    pltpu.sync_copy(idx_hbm.at[pl.ds(pl.multiple_of(wid * RPW, G), RPW)], idx_v)

    def body(g, _):
        chunk = wid * GPW + g
        row0 = pl.multiple_of(chunk * G, G)
        goff = pl.multiple_of(g * G, G)

        @pl.when(chunk < NFULL)
        def _full():
            pltpu.async_copy(table_v.at[idx_v.at[pl.ds(goff, G)]],
                             rows_v, sem).wait()
            pltpu.sync_copy(rows_v, out_hbm.at[pl.ds(row0, G)])

        @pl.when(chunk == NFULL)
        def _partial():
            pltpu.async_copy(table_v.at[idx_v.at[pl.ds(goff, G)]],
                             rows_v, sem).wait()
            pltpu.sync_copy(rows_v.at[pl.ds(0, TAIL)],
                            out_hbm.at[pl.ds(row0, TAIL)])

        return ()

    lax.fori_loop(0, GPW, body, ())


def kernel(x, table):
    idx = x[:, 0].astype(jnp.int32)
    idx = jnp.pad(idx, (0, NPAD - N))
    return _emb_lookup(idx, table)


# speedup vs baseline: 5.6662x; 1.2867x over previous
import functools

import jax
import jax.numpy as jnp
from jax import lax
from jax.experimental import pallas as pl
from jax.experimental.pallas import tpu as pltpu
from jax.experimental.pallas import tpu_sc as plsc

N = 100000
D = 128
G = 128
NC, NS = 2, 16
NW = NC * NS
GPW = 25
RPW = GPW * G
NPAD = NW * RPW
NFULL = N // G
TAIL = N - NFULL * G

_mesh = plsc.VectorSubcoreMesh(core_axis_name="c", subcore_axis_name="s")


@functools.partial(
    pl.kernel,
    mesh=_mesh,
    out_type=[
        jax.ShapeDtypeStruct((N, D), jnp.float32),
        jax.ShapeDtypeStruct((G, D), jnp.float32),
    ],
    scratch_types=[
        pltpu.VMEM((RPW,), jnp.int32),
        pltpu.VMEM((2, G, D), jnp.float32),
        pltpu.VMEM_SHARED((28, D), jnp.float32),
        pltpu.SemaphoreType.DMA,
        pltpu.SemaphoreType.DMA,
    ],
)
def _emb_lookup(idx_hbm, table_hbm, out_hbm, trash_hbm,
                idx_v, rows_v, table_v, gsem, ssem):
    sid = lax.axis_index("s")
    wid = sid * NC + lax.axis_index("c")

    @pl.when(sid == 0)
    def _stage_table():
        pltpu.sync_copy(table_hbm, table_v)

    plsc.subcore_barrier()

    pltpu.sync_copy(idx_hbm.at[pl.ds(pl.multiple_of(wid * RPW, G), RPW)], idx_v)

    def gather(g, buf):
        goff = pl.multiple_of(g * G, G)
        pltpu.async_copy(table_v.at[idx_v.at[pl.ds(goff, G)]], buf, gsem)

    gather(0, rows_v.at[0])

    def body(g, _):
        pb = lax.rem(g, 2)
        cur = rows_v.at[pb]
        nxt = rows_v.at[1 - pb]

        pltpu.make_async_copy(trash_hbm, cur, gsem).wait()

        @pl.when(g >= 1)
        def _drain_store():
            pltpu.make_async_copy(trash_hbm, nxt, ssem).wait()

        @pl.when(g + 1 < GPW)
        def _next_gather():
            gather(g + 1, nxt)

        chunk = wid * GPW + g
        row0 = pl.multiple_of(chunk * G, G)

        @pl.when(chunk < NFULL)
        def _full():
            pltpu.async_copy(cur, out_hbm.at[pl.ds(row0, G)], ssem)

        @pl.when(chunk == NFULL)
        def _partial():
            pltpu.async_copy(cur.at[pl.ds(0, TAIL)],
                             out_hbm.at[pl.ds(row0, TAIL)], ssem)
            pltpu.async_copy(cur.at[pl.ds(TAIL, G - TAIL)],
                             trash_hbm.at[pl.ds(0, G - TAIL)], ssem)

        @pl.when(chunk > NFULL)
        def _idle():
            pltpu.async_copy(cur, trash_hbm, ssem)

        return ()

    lax.fori_loop(0, GPW, body, ())

    pltpu.make_async_copy(trash_hbm, rows_v.at[0], ssem).wait()


def kernel(x, table):
    idx = x[:, 0].astype(jnp.int32)
    idx = jnp.pad(idx, (0, NPAD - N))
    return _emb_lookup(idx, table)[0]


# speedup vs baseline: 5.6666x; 1.0001x over previous
import functools

import jax
import jax.numpy as jnp
from jax import lax
from jax.experimental import pallas as pl
from jax.experimental.pallas import tpu as pltpu
from jax.experimental.pallas import tpu_sc as plsc

N = 100000
D = 128
G = 128
NC, NS = 2, 16
NW = NC * NS
GPW = 25
RPW = GPW * G
NPAD = NW * RPW
NFULL = N // G
TAIL = N - NFULL * G

_mesh = plsc.VectorSubcoreMesh(core_axis_name="c", subcore_axis_name="s")


@functools.partial(
    pl.kernel,
    mesh=_mesh,
    out_type=[
        jax.ShapeDtypeStruct((N, D), jnp.float32),
        jax.ShapeDtypeStruct((G, D), jnp.float32),
    ],
    scratch_types=[
        pltpu.VMEM((RPW,), jnp.int32),
        pltpu.VMEM((3, G, D), jnp.float32),
        pltpu.VMEM_SHARED((28, D), jnp.float32),
        pltpu.SemaphoreType.DMA,
        pltpu.SemaphoreType.DMA,
    ],
)
def _emb_lookup(idx_hbm, table_hbm, out_hbm, trash_hbm,
                idx_v, rows_v, table_v, gsem, ssem):
    sid = lax.axis_index("s")
    wid = sid * NC + lax.axis_index("c")

    @pl.when(sid == 0)
    def _stage_table():
        pltpu.sync_copy(table_hbm, table_v)

    plsc.subcore_barrier()

    pltpu.sync_copy(idx_hbm.at[pl.ds(pl.multiple_of(wid * RPW, G), RPW)], idx_v)

    def gather(g, buf):
        goff = pl.multiple_of(g * G, G)
        pltpu.async_copy(table_v.at[idx_v.at[pl.ds(goff, G)]], buf, gsem)

    gather(0, rows_v.at[0])

    def body(g, _):
        pb = lax.rem(g, 3)
        cur = rows_v.at[pb]
        nxt = rows_v.at[lax.rem(g + 1, 3)]

        pltpu.make_async_copy(trash_hbm, cur, gsem).wait()

        @pl.when(g >= 2)
        def _drain_store():
            pltpu.make_async_copy(trash_hbm, nxt, ssem).wait()

        @pl.when(g + 1 < GPW)
        def _next_gather():
            gather(g + 1, nxt)

        chunk = wid * GPW + g
        row0 = pl.multiple_of(chunk * G, G)

        @pl.when(chunk < NFULL)
        def _full():
            pltpu.async_copy(cur, out_hbm.at[pl.ds(row0, G)], ssem)

        @pl.when(chunk == NFULL)
        def _partial():
            pltpu.async_copy(cur.at[pl.ds(0, TAIL)],
                             out_hbm.at[pl.ds(row0, TAIL)], ssem)
            pltpu.async_copy(cur.at[pl.ds(TAIL, G - TAIL)],
                             trash_hbm.at[pl.ds(0, G - TAIL)], ssem)

        @pl.when(chunk > NFULL)
        def _idle():
            pltpu.async_copy(cur, trash_hbm, ssem)

        return ()

    lax.fori_loop(0, GPW, body, ())

    pltpu.make_async_copy(trash_hbm, rows_v.at[0], ssem).wait()
    pltpu.make_async_copy(trash_hbm, rows_v.at[1], ssem).wait()


def kernel(x, table):
    idx = x[:, 0].astype(jnp.int32)
    idx = jnp.pad(idx, (0, NPAD - N))
    return _emb_lookup(idx, table)[0]
